# baseline (speedup 1.0000x reference)
import dataclasses
import functools

import jax
import jax.numpy as jnp
from jax import lax
from jax.experimental import pallas as pl
from jax.experimental.pallas import tpu as pltpu
from jax.experimental.pallas import tpu_sc as plsc

N = 65536
NNZ = 4194304
E = 100
EP = 128
LAYERS = 3

NB = 8
RB = N // NB
NW = 32
EPW = NNZ // NW
BLK = 512
WCAP = (EPW // BLK + 1) * BLK
W = 128
ZR = 64

_mesh = plsc.VectorSubcoreMesh(core_axis_name="c", subcore_axis_name="s")

_sc_params = pltpu.CompilerParams()
if "needs_layout_passes" in pltpu.CompilerParams.__dataclass_fields__:
    _sc_params = dataclasses.replace(_sc_params, needs_layout_passes=False)


def _mm_body(h_ref, w_ref, b_ref, o_ref):
    o_ref[...] = (
        jnp.dot(h_ref[...], w_ref[...], preferred_element_type=jnp.float32)
        + b_ref[...]
    )


def _mm(h, w, b):
    return pl.pallas_call(
        _mm_body,
        grid=(64,),
        in_specs=[
            pl.BlockSpec((N // 64, EP), lambda i: (i, 0)),
            pl.BlockSpec((EP, EP), lambda i: (0, 0)),
            pl.BlockSpec((1, EP), lambda i: (0, 0)),
        ],
        out_specs=pl.BlockSpec((N // 64, EP), lambda i: (i, 0)),
        out_shape=jax.ShapeDtypeStruct((N, EP), jnp.float32),
    )(h, w, b)


def _final_body(e_ref, h1_ref, h2_ref, h3_ref, o_ref):
    o_ref[...] = 0.25 * (e_ref[...] + h1_ref[...] + h2_ref[...] + h3_ref[...])


def _final(emb_p, h1, h2, h3):
    spec = pl.BlockSpec((N // 64, EP), lambda i: (i, 0))
    return pl.pallas_call(
        _final_body,
        grid=(64,),
        in_specs=[spec, spec, spec, spec],
        out_specs=spec,
        out_shape=jax.ShapeDtypeStruct((N, EP), jnp.float32),
    )(emb_p, h1, h2, h3)


def _partition_xla(adj_indices, adj_values):
    dst = adj_indices[0]
    src = adj_indices[1]
    e = jnp.arange(NNZ, dtype=jnp.int32)
    wk = e >> 17
    b = dst >> 13
    key = wk * NB + b
    order = jnp.argsort(key, stable=True)
    key_s = key[order]
    counts = jnp.zeros((NW * NB,), jnp.int32).at[key].add(1)
    starts = jnp.concatenate(
        [jnp.zeros((1,), jnp.int32), jnp.cumsum(counts)[:-1].astype(jnp.int32)]
    )
    pos = e - starts[key_s]
    packed = (src | ((dst & (RB - 1)) << 16))[order]
    slot = key_s * WCAP + pos
    idxp = jnp.zeros((NW * NB * WCAP,), jnp.int32).at[slot].set(packed)
    valp = jnp.zeros((NW * NB * WCAP,), jnp.float32).at[slot].set(
        adj_values[order]
    )
    nblk = (counts + BLK - 1) // BLK
    cnt_rep = jnp.repeat(nblk[:, None], 16, axis=1)
    return (
        idxp.reshape(NW * NB, WCAP),
        valp.reshape(NW * NB, WCAP),
        cnt_rep,
    )


@functools.partial(
    pl.kernel,
    out_type=jax.ShapeDtypeStruct((N, EP), jnp.float32),
    mesh=_mesh,
    compiler_params=_sc_params,
    scratch_types=[
        pltpu.VMEM_SHARED((RB, EP), jnp.float32),
        pltpu.VMEM((NW * NB, 16), jnp.int32),
        pltpu.VMEM((BLK,), jnp.int32),
        pltpu.VMEM((BLK,), jnp.float32),
        pltpu.VMEM((W,), jnp.int32),
        pltpu.VMEM((W,), jnp.int32),
        pltpu.VMEM((W, EP), jnp.float32),
        pltpu.VMEM((ZR, EP), jnp.float32),
        pltpu.SemaphoreType.DMA,
    ],
)
def _agg_kernel(hx_hbm, idxp_hbm, valp_hbm, cnt_hbm, out_hbm,
                accum, cnt_v, idx_v, val_v, src_v, ldst_v, rows_v, zbuf, sem):
    c = lax.axis_index("c")
    s = lax.axis_index("s")
    stripe = RB // 16

    pltpu.sync_copy(cnt_hbm, cnt_v)

    @pl.loop(0, ZR)
    def _(r):
        for j in range(EP // 16):
            zbuf[r, pl.ds(16 * j, 16)] = jnp.zeros((16,), jnp.float32)

    for rnd in range(NB // 2):
        bkt = 2 * rnd + c

        @pl.loop(0, stripe // ZR)
        def _(k):
            pltpu.sync_copy(zbuf, accum.at[pl.ds(s * stripe + k * ZR, ZR)])

        plsc.subcore_barrier()

        for fi in range(2):
            wkr = s + 16 * fi
            row = wkr * NB + bkt
            nblk = jnp.max(cnt_v[row])

            @pl.loop(0, nblk)
            def _(blk):
                pltpu.sync_copy(idxp_hbm.at[row, pl.ds(blk * BLK, BLK)], idx_v)
                pltpu.sync_copy(valp_hbm.at[row, pl.ds(blk * BLK, BLK)], val_v)
                for wi in range(BLK // W):
                    @pl.loop(0, W // 16)
                    def _(k):
                        p = idx_v[pl.ds(wi * W + k * 16, 16)]
                        src_v[pl.ds(k * 16, 16)] = p & 0xFFFF
                        ldst_v[pl.ds(k * 16, 16)] = lax.shift_right_logical(
                            p, 16)

                    pltpu.async_copy(hx_hbm.at[src_v], rows_v, sem).wait()

                    @pl.loop(0, W)
                    def _(ei):
                        vv = plsc.load_gather(
                            val_v,
                            [jnp.full((16,), wi * W + ei, jnp.int32)])
                        for j in range(EP // 16):
                            rows_v[ei, pl.ds(16 * j, 16)] = (
                                rows_v[ei, pl.ds(16 * j, 16)] * vv)

                    pltpu.sync_copy(rows_v, accum.at[ldst_v], add=True)

        plsc.subcore_barrier()
        pltpu.sync_copy(
            accum.at[pl.ds(s * stripe, stripe)],
            out_hbm.at[pl.ds(bkt * RB + s * stripe, stripe)])
        plsc.subcore_barrier()


def kernel(adj_indices, adj_values, embedding, W0, W1, W2, b0, b1, b2):
    emb_p = jnp.pad(embedding, ((0, 0), (0, EP - E)))
    Ws = [jnp.pad(w, ((0, EP - E), (0, EP - E))) for w in (W0, W1, W2)]
    bs = [jnp.pad(b, (0, EP - E)).reshape(1, EP) for b in (b0, b1, b2)]

    idxp, valp, cnt = _partition_xla(adj_indices, adj_values)

    h = emb_p
    hs = []
    for i in range(LAYERS):
        hx = _mm(h, Ws[i], bs[i])
        h = _agg_kernel(hx, idxp, valp, cnt)
        hs.append(h)

    out_p = _final(emb_p, hs[0], hs[1], hs[2])
    return out_p[:, :E]


# speedup vs baseline: 5.9418x; 5.9418x over previous
import dataclasses
import functools

import jax
import jax.numpy as jnp
from jax import lax
from jax.experimental import pallas as pl
from jax.experimental.pallas import tpu as pltpu
from jax.experimental.pallas import tpu_sc as plsc

N = 65536
NNZ = 4194304
E = 100
EP = 128
LAYERS = 3

NB = 8
RB = N // NB
NW = 32
EPW = NNZ // NW
BLK = 512
WCAP = (EPW // BLK + 1) * BLK
W = 128
ZR = 64

_mesh = plsc.VectorSubcoreMesh(core_axis_name="c", subcore_axis_name="s")

_sc_params = pltpu.CompilerParams()
if "needs_layout_passes" in pltpu.CompilerParams.__dataclass_fields__:
    _sc_params = dataclasses.replace(_sc_params, needs_layout_passes=False)


def _mm_body(h_ref, w_ref, b_ref, o_ref):
    o_ref[...] = (
        jnp.dot(h_ref[...], w_ref[...], preferred_element_type=jnp.float32)
        + b_ref[...]
    )


def _mm(h, w, b):
    return pl.pallas_call(
        _mm_body,
        grid=(64,),
        in_specs=[
            pl.BlockSpec((N // 64, EP), lambda i: (i, 0)),
            pl.BlockSpec((EP, EP), lambda i: (0, 0)),
            pl.BlockSpec((1, EP), lambda i: (0, 0)),
        ],
        out_specs=pl.BlockSpec((N // 64, EP), lambda i: (i, 0)),
        out_shape=jax.ShapeDtypeStruct((N, EP), jnp.float32),
    )(h, w, b)


def _final_body(e_ref, h1_ref, h2_ref, h3_ref, o_ref):
    o_ref[...] = 0.25 * (e_ref[...] + h1_ref[...] + h2_ref[...] + h3_ref[...])


def _final(emb_p, h1, h2, h3):
    spec = pl.BlockSpec((N // 64, EP), lambda i: (i, 0))
    return pl.pallas_call(
        _final_body,
        grid=(64,),
        in_specs=[spec, spec, spec, spec],
        out_specs=spec,
        out_shape=jax.ShapeDtypeStruct((N, EP), jnp.float32),
    )(emb_p, h1, h2, h3)


def _partition_xla(adj_indices, adj_values):
    dst = adj_indices[0]
    src = adj_indices[1]
    e = jnp.arange(NNZ, dtype=jnp.int32)
    wk = e >> 17
    b = dst >> 13
    key = wk * NB + b
    order = jnp.argsort(key, stable=True)
    key_s = key[order]
    counts = jnp.zeros((NW * NB,), jnp.int32).at[key].add(1)
    starts = jnp.concatenate(
        [jnp.zeros((1,), jnp.int32), jnp.cumsum(counts)[:-1].astype(jnp.int32)]
    )
    pos = e - starts[key_s]
    packed = (src | ((dst & (RB - 1)) << 16))[order]
    slot = key_s * WCAP + pos
    idxp = jnp.zeros((NW * NB * WCAP,), jnp.int32).at[slot].set(packed)
    valp = jnp.zeros((NW * NB * WCAP,), jnp.float32).at[slot].set(
        adj_values[order]
    )
    nblk = (counts + BLK - 1) // BLK
    cnt_rep = jnp.repeat(nblk[:, None], 16, axis=1)
    return idxp, valp, cnt_rep


PW = 512
SLAB = 1040
SCAP = 2 * SLAB


@functools.partial(
    pl.kernel,
    out_type=[
        jax.ShapeDtypeStruct((NW * NB * WCAP,), jnp.int32),
        jax.ShapeDtypeStruct((NW * NB * WCAP,), jnp.float32),
        jax.ShapeDtypeStruct((NW * NB, 16), jnp.int32),
    ],
    mesh=_mesh,
    compiler_params=_sc_params,
    scratch_types=[
        pltpu.VMEM((PW,), jnp.int32),
        pltpu.VMEM((PW,), jnp.int32),
        pltpu.VMEM((PW,), jnp.float32),
        pltpu.VMEM((NB * SCAP,), jnp.int32),
        pltpu.VMEM((NB * SCAP,), jnp.float32),
        pltpu.VMEM((NB, 16), jnp.int32),
    ],
)
def _part_kernel(dst_hbm, src_hbm, val_hbm, idxp_hbm, valp_hbm, cnt_hbm,
                 dst_v, srcw_v, valw_v, sidx, sval, cnt_v):
    c = lax.axis_index("c")
    s = lax.axis_index("s")
    wkr = s + 16 * c
    base = wkr * EPW

    zero16i = jnp.zeros((16,), jnp.int32)
    zero16f = jnp.zeros((16,), jnp.float32)
    lane = lax.iota(jnp.int32, 16)

    init = (jnp.int32(0),) * (3 * NB)

    @pl.loop(0, EPW // PW, init_carry=init)
    def carry(t, st):
        fs = list(st[:NB])
        ns = list(st[NB:2 * NB])
        hs = list(st[2 * NB:])
        off = base + t * PW
        pltpu.sync_copy(dst_hbm.at[pl.ds(off, PW)], dst_v)
        pltpu.sync_copy(src_hbm.at[pl.ds(off, PW)], srcw_v)
        pltpu.sync_copy(val_hbm.at[pl.ds(off, PW)], valw_v)

        for b_i in range(NB):
            full = fs[b_i] >= BLK
            hb = pl.multiple_of(b_i * SCAP + hs[b_i] * SLAB, 8)
            ho = pl.multiple_of(b_i * SCAP + (1 - hs[b_i]) * SLAB, 8)

            @pl.when(full)
            def _():
                go = pl.multiple_of(
                    (wkr * NB + b_i) * WCAP + ns[b_i] * BLK, 128)
                pltpu.sync_copy(sidx.at[pl.ds(hb, BLK)],
                                idxp_hbm.at[pl.ds(go, BLK)])
                pltpu.sync_copy(sval.at[pl.ds(hb, BLK)],
                                valp_hbm.at[pl.ds(go, BLK)])

                @pl.loop(0, (fs[b_i] - BLK + 15) // 16)
                def _(j):
                    sidx[pl.ds(ho + j * 16, 16)] = (
                        sidx[pl.ds(hb + BLK + j * 16, 16)])
                    sval[pl.ds(ho + j * 16, 16)] = (
                        sval[pl.ds(hb + BLK + j * 16, 16)])

            fi = full.astype(jnp.int32)
            fs[b_i] = fs[b_i] - BLK * fi
            ns[b_i] = ns[b_i] + fi
            hs[b_i] = hs[b_i] ^ fi

        @pl.loop(0, PW // 16, init_carry=tuple(fs))
        def inner(k, fc):
            d = dst_v[pl.ds(k * 16, 16)]
            p = srcw_v[pl.ds(k * 16, 16)] | ((d & (RB - 1)) << 16)
            v = valw_v[pl.ds(k * 16, 16)]
            bk = lax.shift_right_logical(d, 13)
            out = []
            for b_i in range(NB):
                f = fc[b_i]
                pos = b_i * SCAP + hs[b_i] * SLAB + f
                m = bk == b_i
                plsc.store_compressed(sidx.at[pl.ds(pos, 16)], p, mask=m)
                plsc.store_compressed(sval.at[pl.ds(pos, 16)], v, mask=m)
                out.append(f + jnp.max(plsc.all_reduce_population_count(m)))
            return tuple(out)

        return tuple(inner) + tuple(ns) + tuple(hs)

    fs = list(carry[:NB])
    ns = list(carry[NB:2 * NB])
    hs = list(carry[2 * NB:])

    for b_i in range(NB):
        hb = b_i * SCAP + hs[b_i] * SLAB

        @pl.when(fs[b_i] > 0)
        def _():
            rem_lo = (fs[b_i] // BLK) * BLK
            rem = fs[b_i] - rem_lo

            @pl.when(rem > 0)
            def _():
                @pl.loop(0, BLK // 16)
                def _(j):
                    keep = (lane + j * 16) < rem
                    q = hb + rem_lo + j * 16
                    cur_i = sidx[pl.ds(q, 16)]
                    cur_v = sval[pl.ds(q, 16)]
                    sidx[pl.ds(q, 16)] = jnp.where(keep, cur_i, zero16i)
                    sval[pl.ds(q, 16)] = jnp.where(keep, cur_v, zero16f)

    pl.delay(2000)

    for b_i in range(NB):
        hb = pl.multiple_of(b_i * SCAP + hs[b_i] * SLAB, 8)

        @pl.when(fs[b_i] >= BLK)
        def _():
            go = pl.multiple_of(
                (wkr * NB + b_i) * WCAP + ns[b_i] * BLK, 128)
            pltpu.sync_copy(sidx.at[pl.ds(hb, BLK)],
                            idxp_hbm.at[pl.ds(go, BLK)])
            pltpu.sync_copy(sval.at[pl.ds(hb, BLK)],
                            valp_hbm.at[pl.ds(go, BLK)])

        nfull = fs[b_i] // BLK
        rem_lo = pl.multiple_of(nfull * BLK, 8)
        rem = fs[b_i] - rem_lo

        @pl.when(rem > 0)
        def _():
            go = pl.multiple_of(
                (wkr * NB + b_i) * WCAP + (ns[b_i] + nfull) * BLK, 128)
            pltpu.sync_copy(sidx.at[pl.ds(hb + rem_lo, BLK)],
                            idxp_hbm.at[pl.ds(go, BLK)])
            pltpu.sync_copy(sval.at[pl.ds(hb + rem_lo, BLK)],
                            valp_hbm.at[pl.ds(go, BLK)])

        ns[b_i] = ns[b_i] + nfull + (rem > 0).astype(jnp.int32)
        cnt_v[b_i, :] = jnp.full((16,), ns[b_i], jnp.int32)

    pltpu.sync_copy(cnt_v, cnt_hbm.at[pl.ds(wkr * NB, NB)])


@functools.partial(
    pl.kernel,
    out_type=jax.ShapeDtypeStruct((N, EP), jnp.float32),
    mesh=_mesh,
    compiler_params=_sc_params,
    scratch_types=[
        pltpu.VMEM_SHARED((RB, EP), jnp.float32),
        pltpu.VMEM((NW * NB, 16), jnp.int32),
        pltpu.VMEM((BLK,), jnp.int32),
        pltpu.VMEM((BLK,), jnp.float32),
        pltpu.VMEM((W,), jnp.int32),
        pltpu.VMEM((W,), jnp.int32),
        pltpu.VMEM((W, EP), jnp.float32),
        pltpu.VMEM((ZR, EP), jnp.float32),
        pltpu.SemaphoreType.DMA,
    ],
)
def _agg_kernel(hx_hbm, idxp_hbm, valp_hbm, cnt_hbm, out_hbm,
                accum, cnt_v, idx_v, val_v, src_v, ldst_v, rows_v, zbuf, sem):
    c = lax.axis_index("c")
    s = lax.axis_index("s")
    stripe = RB // 16

    pltpu.sync_copy(cnt_hbm, cnt_v)

    @pl.loop(0, ZR)
    def _(r):
        for j in range(EP // 16):
            zbuf[r, pl.ds(16 * j, 16)] = jnp.zeros((16,), jnp.float32)

    for rnd in range(NB // 2):
        bkt = 2 * rnd + c

        @pl.loop(0, stripe // ZR)
        def _(k):
            pltpu.sync_copy(zbuf, accum.at[pl.ds(s * stripe + k * ZR, ZR)])

        plsc.subcore_barrier()

        for fi in range(2):
            wkr = s + 16 * fi
            row = wkr * NB + bkt
            nblk = jnp.max(cnt_v[row])

            @pl.loop(0, nblk)
            def _(blk):
                go = pl.multiple_of(row * WCAP + blk * BLK, 128)
                pltpu.sync_copy(idxp_hbm.at[pl.ds(go, BLK)], idx_v)
                pltpu.sync_copy(valp_hbm.at[pl.ds(go, BLK)], val_v)
                for wi in range(BLK // W):
                    @pl.loop(0, W // 16)
                    def _(k):
                        p = idx_v[pl.ds(wi * W + k * 16, 16)]
                        src_v[pl.ds(k * 16, 16)] = p & 0xFFFF
                        ldst_v[pl.ds(k * 16, 16)] = lax.shift_right_logical(
                            p, 16)

                    pltpu.async_copy(hx_hbm.at[src_v], rows_v, sem).wait()

                    @pl.loop(0, W)
                    def _(ei):
                        vv = plsc.load_gather(
                            val_v,
                            [jnp.full((16,), wi * W + ei, jnp.int32)])
                        for j in range(EP // 16):
                            rows_v[ei, pl.ds(16 * j, 16)] = (
                                rows_v[ei, pl.ds(16 * j, 16)] * vv)

                    pltpu.sync_copy(rows_v, accum.at[ldst_v], add=True)

        plsc.subcore_barrier()
        pltpu.sync_copy(
            accum.at[pl.ds(s * stripe, stripe)],
            out_hbm.at[pl.ds(bkt * RB + s * stripe, stripe)])
        plsc.subcore_barrier()


def kernel(adj_indices, adj_values, embedding, W0, W1, W2, b0, b1, b2):
    emb_p = jnp.pad(embedding, ((0, 0), (0, EP - E)))
    Ws = [jnp.pad(w, ((0, EP - E), (0, EP - E))) for w in (W0, W1, W2)]
    bs = [jnp.pad(b, (0, EP - E)).reshape(1, EP) for b in (b0, b1, b2)]

    idxp, valp, cnt = _part_kernel(
        adj_indices[0], adj_indices[1], adj_values)

    h = emb_p
    hs = []
    for i in range(LAYERS):
        hx = _mm(h, Ws[i], bs[i])
        h = _agg_kernel(hx, idxp, valp, cnt)
        hs.append(h)

    out_p = _final(emb_p, hs[0], hs[1], hs[2])
    return out_p[:, :E]
